# Initial kernel scaffold; baseline (speedup 1.0000x reference)
import functools

import jax
import jax.numpy as jnp
from jax import lax
from jax.experimental import pallas as pl
from jax.experimental.pallas import tpu as pltpu
from jax.experimental.pallas import tpu_sc as plsc

_B = 4096
_D = 4096
_C = 20000
_E = 256
_M = 128
_NNZ = 163840

_CB = 1000
_NCB = _C // _CB

_NC = 2
_NS = 16
_HALF = _M // _NC
_NNZ_PER = _NNZ // _NS
_CHUNK = 128
_NCHUNK = _NNZ_PER // _CHUNK
_BROWS = _B // _NS


def _mm_body(ce_ref, w_ref, b_ref, mp_ref, out_ref):
    pce = jnp.dot(ce_ref[...], w_ref[...], preferred_element_type=jnp.float32)
    pce = pce + b_ref[...]
    pce = jax.nn.swish(pce)
    pce_t = pce.T
    upd = jnp.dot(pce_t, mp_ref[...], preferred_element_type=jnp.float32)

    @pl.when(pl.program_id(0) == 0)
    def _():
        out_ref[...] = upd

    @pl.when(pl.program_id(0) != 0)
    def _():
        out_ref[...] += upd


def _matmul_pcm_t(card_embeddings, w, bias_row, metapath):
    return pl.pallas_call(
        _mm_body,
        grid=(_NCB,),
        in_specs=[
            pl.BlockSpec((_CB, _E), lambda i: (i, 0)),
            pl.BlockSpec((_E, _M), lambda i: (0, 0)),
            pl.BlockSpec((1, _M), lambda i: (0, 0)),
            pl.BlockSpec((_CB, _D), lambda i: (i, 0)),
        ],
        out_specs=pl.BlockSpec((_M, _D), lambda i: (0, 0)),
        out_shape=jax.ShapeDtypeStruct((_M, _D), jnp.float32),
        compiler_params=pltpu.CompilerParams(
            dimension_semantics=("arbitrary",),
        ),
    )(card_embeddings, w, bias_row, metapath)


def _sc_body(table_hbm, cols_hbm, rows_hbm, vals_hbm, out_hbm,
             cols_v, rows_v, vals_v, gb, zb, acc):
    cid = lax.axis_index("c")
    sid = lax.axis_index("s")

    pltpu.sync_copy(cols_hbm.at[sid], cols_v)
    pltpu.sync_copy(rows_hbm.at[sid], rows_v)
    pltpu.sync_copy(vals_hbm.at[sid], vals_v)

    offv = jnp.full((16,), cid * _D, jnp.int32)

    def _off_body(j, _):
        for k in range(_CHUNK // 16):
            cols_v[j, pl.ds(k * 16, 16)] = cols_v[j, pl.ds(k * 16, 16)] + offv
        return 0

    lax.fori_loop(0, _NCHUNK, _off_body, 0)

    zv = jnp.zeros((16,), jnp.float32)

    def _zero_body(r, _):
        for k in range(_HALF // 16):
            zb[r, pl.ds(k * 16, 16)] = zv
        return 0

    lax.fori_loop(0, _BROWS, _zero_body, 0)
    pltpu.sync_copy(zb, acc.at[pl.ds(sid * _BROWS, _BROWS)])
    plsc.subcore_barrier()

    def _chunk_body(c, _):
        pltpu.sync_copy(table_hbm.at[cols_v.at[c]], gb)
        base = c * _CHUNK

        def _scale_row(r, _):
            idx = jnp.full((16,), base + r, jnp.int32)
            v = plsc.load_gather(vals_v, [idx])
            for k in range(_HALF // 16):
                gb[r, pl.ds(k * 16, 16)] = gb[r, pl.ds(k * 16, 16)] * v
            return 0

        lax.fori_loop(0, _CHUNK, _scale_row, 0)
        pltpu.sync_copy(gb, acc.at[rows_v.at[c]], add=True)
        return 0

    lax.fori_loop(0, _NCHUNK, _chunk_body, 0)
    plsc.subcore_barrier()

    pltpu.sync_copy(acc.at[pl.ds(sid * _BROWS, _BROWS)],
                    out_hbm.at[cid, pl.ds(sid * _BROWS, _BROWS)])


_sc_call = functools.partial(
    pl.kernel,
    out_type=jax.ShapeDtypeStruct((_NC, _B, _HALF), jnp.float32),
    mesh=plsc.VectorSubcoreMesh(core_axis_name="c", subcore_axis_name="s"),
    scratch_types=[
        pltpu.VMEM((_NCHUNK, _CHUNK), jnp.int32),
        pltpu.VMEM((_NCHUNK, _CHUNK), jnp.int32),
        pltpu.VMEM((_NNZ_PER,), jnp.float32),
        pltpu.VMEM((_CHUNK, _HALF), jnp.float32),
        pltpu.VMEM((_BROWS, _HALF), jnp.float32),
        pltpu.VMEM_SHARED((_B, _HALF), jnp.float32),
    ],
)(_sc_body)


def kernel(pool_values, card_embeddings, metapath, kernel, bias, pool_rows, pool_cols):
    bias_row = bias.reshape(1, _M)
    pcm_t = _matmul_pcm_t(card_embeddings, kernel, bias_row, metapath)
    table = (
        pcm_t.reshape(_NC, _HALF, _D).transpose(0, 2, 1).reshape(_NC * _D, _HALF)
    )
    cols = pool_cols.astype(jnp.int32).reshape(_NS, _NCHUNK, _CHUNK)
    rows = pool_rows.astype(jnp.int32).reshape(_NS, _NCHUNK, _CHUNK)
    vals = pool_values.reshape(_NS, _NNZ_PER)
    out3 = _sc_call(table, cols, rows, vals)
    return out3.transpose(1, 0, 2).reshape(_B, _M)


# speedup vs baseline: 3.1611x; 3.1611x over previous
import functools

import jax
import jax.numpy as jnp
from jax import lax
from jax.experimental import pallas as pl
from jax.experimental.pallas import tpu as pltpu
from jax.experimental.pallas import tpu_sc as plsc

_B = 4096
_D = 4096
_C = 20000
_E = 256
_M = 128
_NNZ = 163840

_CB = 1000
_NCB = _C // _CB

_NC = 2
_NS = 16
_HALF = _M // _NC
_NNZ_PER = _NNZ // _NS
_CHUNK = 128
_NCHUNK = _NNZ_PER // _CHUNK
_BROWS = _B // _NS


def _mm_body(ce_ref, w_ref, b_ref, mp_ref, out_ref):
    pce = jnp.dot(ce_ref[...], w_ref[...], preferred_element_type=jnp.float32)
    pce = pce + b_ref[...]
    pce = jax.nn.swish(pce)
    pce_t = pce.T
    upd = jnp.dot(pce_t, mp_ref[...], preferred_element_type=jnp.float32)

    @pl.when(pl.program_id(0) == 0)
    def _():
        out_ref[...] = upd

    @pl.when(pl.program_id(0) != 0)
    def _():
        out_ref[...] += upd


def _matmul_pcm_t(card_embeddings, w, bias_row, metapath):
    return pl.pallas_call(
        _mm_body,
        grid=(_NCB,),
        in_specs=[
            pl.BlockSpec((_CB, _E), lambda i: (i, 0)),
            pl.BlockSpec((_E, _M), lambda i: (0, 0)),
            pl.BlockSpec((1, _M), lambda i: (0, 0)),
            pl.BlockSpec((_CB, _D), lambda i: (i, 0)),
        ],
        out_specs=pl.BlockSpec((_M, _D), lambda i: (0, 0)),
        out_shape=jax.ShapeDtypeStruct((_M, _D), jnp.float32),
        compiler_params=pltpu.CompilerParams(
            dimension_semantics=("arbitrary",),
        ),
    )(card_embeddings, w, bias_row, metapath)


def _sc_body(table_hbm, cols_hbm, rows_hbm, vals_hbm, out_hbm,
             cols_v, rows_v, vals_v, gb, zb, acc):
    cid = lax.axis_index("c")
    sid = lax.axis_index("s")

    pltpu.sync_copy(cols_hbm.at[sid], cols_v)
    pltpu.sync_copy(rows_hbm.at[sid], rows_v)
    pltpu.sync_copy(vals_hbm.at[sid], vals_v)

    offv = jnp.full((16,), cid * _D, jnp.int32)

    def _off_body(j, _):
        for k in range(_CHUNK // 16):
            cols_v[j, pl.ds(k * 16, 16)] = cols_v[j, pl.ds(k * 16, 16)] + offv
        return 0

    lax.fori_loop(0, _NCHUNK, _off_body, 0)

    zv = jnp.zeros((16,), jnp.float32)

    def _zero_body(r, _):
        for k in range(_HALF // 16):
            zb[r, pl.ds(k * 16, 16)] = zv
        return 0

    lax.fori_loop(0, _BROWS, _zero_body, 0)
    pltpu.sync_copy(zb, acc.at[pl.ds(sid * _BROWS, _BROWS)])
    plsc.subcore_barrier()

    def _chunk_body(c, _):
        pltpu.sync_copy(table_hbm.at[cols_v.at[c]], gb)
        base = c * _CHUNK

        def _scale_grp(g, _):
            vg = vals_v[pl.ds(pl.multiple_of(base + g * 16, 16), 16)]
            for l in range(16):
                v = vg.at[jnp.full((16,), l, jnp.int32)].get(
                    mode="promise_in_bounds")
                r = g * 16 + l
                for k in range(_HALF // 16):
                    gb[r, pl.ds(k * 16, 16)] = gb[r, pl.ds(k * 16, 16)] * v
            return 0

        lax.fori_loop(0, _CHUNK // 16, _scale_grp, 0)
        pltpu.sync_copy(gb, acc.at[rows_v.at[c]], add=True)
        return 0

    lax.fori_loop(0, _NCHUNK, _chunk_body, 0)
    plsc.subcore_barrier()

    pltpu.sync_copy(acc.at[pl.ds(sid * _BROWS, _BROWS)],
                    out_hbm.at[cid, pl.ds(sid * _BROWS, _BROWS)])


_sc_call = functools.partial(
    pl.kernel,
    out_type=jax.ShapeDtypeStruct((_NC, _B, _HALF), jnp.float32),
    mesh=plsc.VectorSubcoreMesh(core_axis_name="c", subcore_axis_name="s"),
    scratch_types=[
        pltpu.VMEM((_NCHUNK, _CHUNK), jnp.int32),
        pltpu.VMEM((_NCHUNK, _CHUNK), jnp.int32),
        pltpu.VMEM((_NNZ_PER,), jnp.float32),
        pltpu.VMEM((_CHUNK, _HALF), jnp.float32),
        pltpu.VMEM((_BROWS, _HALF), jnp.float32),
        pltpu.VMEM_SHARED((_B, _HALF), jnp.float32),
    ],
    compiler_params=pltpu.CompilerParams(use_tc_tiling_on_sc=False),
)(_sc_body)


def kernel(pool_values, card_embeddings, metapath, kernel, bias, pool_rows, pool_cols):
    bias_row = bias.reshape(1, _M)
    pcm_t = _matmul_pcm_t(card_embeddings, kernel, bias_row, metapath)
    table = (
        pcm_t.reshape(_NC, _HALF, _D).transpose(0, 2, 1).reshape(_NC * _D, _HALF)
    )
    cols = pool_cols.astype(jnp.int32).reshape(_NS, _NCHUNK, _CHUNK)
    rows = pool_rows.astype(jnp.int32).reshape(_NS, _NCHUNK, _CHUNK)
    vals = pool_values.reshape(_NS, _NNZ_PER)
    out3 = _sc_call(table, cols, rows, vals)
    return out3.transpose(1, 0, 2).reshape(_B, _M)


# speedup vs baseline: 5.7889x; 1.8313x over previous
import functools

import jax
import jax.numpy as jnp
from jax import lax
from jax.experimental import pallas as pl
from jax.experimental.pallas import tpu as pltpu
from jax.experimental.pallas import tpu_sc as plsc

_B = 4096
_D = 4096
_C = 20000
_E = 256
_M = 128
_NNZ = 163840

_CB = 1000
_NCB = _C // _CB

_NC = 2
_NS = 16
_HALF = _M // _NC
_NNZ_PER = _NNZ // _NS
_CHUNK = 128
_NCHUNK = _NNZ_PER // _CHUNK
_BROWS = _B // _NS


def _mm_body(ce_ref, w_ref, b_ref, mp_ref, out_ref):
    pce = jnp.dot(ce_ref[...], w_ref[...], preferred_element_type=jnp.float32)
    pce = pce + b_ref[...]
    pce = jax.nn.swish(pce)
    pce_t = pce.T
    upd = jnp.dot(pce_t, mp_ref[...], preferred_element_type=jnp.float32)

    @pl.when(pl.program_id(0) == 0)
    def _():
        out_ref[...] = upd

    @pl.when(pl.program_id(0) != 0)
    def _():
        out_ref[...] += upd


def _matmul_pcm_t(card_embeddings, w, bias_row, metapath):
    return pl.pallas_call(
        _mm_body,
        grid=(_NCB,),
        in_specs=[
            pl.BlockSpec((_CB, _E), lambda i: (i, 0)),
            pl.BlockSpec((_E, _M), lambda i: (0, 0)),
            pl.BlockSpec((1, _M), lambda i: (0, 0)),
            pl.BlockSpec((_CB, _D), lambda i: (i, 0)),
        ],
        out_specs=pl.BlockSpec((_M, _D), lambda i: (0, 0)),
        out_shape=jax.ShapeDtypeStruct((_M, _D), jnp.float32),
        compiler_params=pltpu.CompilerParams(
            dimension_semantics=("arbitrary",),
        ),
    )(card_embeddings, w, bias_row, metapath)


def _sc_body(table_hbm, cols_hbm, rows_hbm, vals_hbm, out_hbm,
             cols_v, rows_v, vals_v, gb, zb, acc, gsem):
    cid = lax.axis_index("c")
    sid = lax.axis_index("s")

    pltpu.sync_copy(cols_hbm.at[sid], cols_v)
    pltpu.sync_copy(rows_hbm.at[sid], rows_v)
    pltpu.sync_copy(vals_hbm.at[sid], vals_v)

    offv = jnp.full((16,), cid * _D, jnp.int32)

    def _off_body(j, _):
        for k in range(_CHUNK // 16):
            cols_v[j, pl.ds(k * 16, 16)] = cols_v[j, pl.ds(k * 16, 16)] + offv
        return 0

    lax.fori_loop(0, _NCHUNK, _off_body, 0)

    zv = jnp.zeros((16,), jnp.float32)

    def _zero_body(r, _):
        for k in range(_HALF // 16):
            zb[r, pl.ds(k * 16, 16)] = zv
        return 0

    lax.fori_loop(0, _BROWS, _zero_body, 0)
    pltpu.sync_copy(zb, acc.at[pl.ds(sid * _BROWS, _BROWS)])
    plsc.subcore_barrier()

    iota16 = lax.iota(jnp.int32, 16)

    def _fire(c, b):
        return pltpu.async_copy(table_hbm.at[cols_v.at[c]], gb.at[b], gsem)

    _fire(0, 0)

    def _chunk_pair(i, _):
        for b in (0, 1):
            c = 2 * i + b
            pltpu.make_async_copy(table_hbm.at[cols_v.at[c]], gb.at[b],
                                  gsem).wait()

            @pl.when(c + 1 < _NCHUNK)
            def _():
                _fire(c + 1, 1 - b)

            base = c * _CHUNK
            gbb = gb.at[b]

            @plsc.parallel_loop(0, _CHUNK // 16, 1, unroll=2)
            def _blk(g):
                vg = vals_v[pl.ds(pl.multiple_of(base + g * 16, 16), 16)]
                for l in range(16):
                    v = vg.at[jnp.full((16,), l, jnp.int32)].get(
                        mode="promise_in_bounds")
                    r = g * 16 + l
                    for k in range(_HALF // 16):
                        gbb[r, pl.ds(k * 16, 16)] = gbb[r, pl.ds(k * 16, 16)] * v
            pltpu.sync_copy(gb.at[b], acc.at[rows_v.at[c]], add=True)
        return 0

    lax.fori_loop(0, _NCHUNK // 2, _chunk_pair, 0)
    plsc.subcore_barrier()

    pltpu.sync_copy(acc.at[pl.ds(sid * _BROWS, _BROWS)],
                    out_hbm.at[cid, pl.ds(sid * _BROWS, _BROWS)])


_sc_call = functools.partial(
    pl.kernel,
    out_type=jax.ShapeDtypeStruct((_NC, _B, _HALF), jnp.float32),
    mesh=plsc.VectorSubcoreMesh(core_axis_name="c", subcore_axis_name="s"),
    scratch_types=[
        pltpu.VMEM((_NCHUNK, _CHUNK), jnp.int32),
        pltpu.VMEM((_NCHUNK, _CHUNK), jnp.int32),
        pltpu.VMEM((_NNZ_PER,), jnp.float32),
        pltpu.VMEM((2, _CHUNK, _HALF), jnp.float32),
        pltpu.VMEM((_BROWS, _HALF), jnp.float32),
        pltpu.VMEM_SHARED((_B, _HALF), jnp.float32),
        pltpu.SemaphoreType.DMA,
    ],
    compiler_params=pltpu.CompilerParams(use_tc_tiling_on_sc=False),
)(_sc_body)


def kernel(pool_values, card_embeddings, metapath, kernel, bias, pool_rows, pool_cols):
    bias_row = bias.reshape(1, _M)
    pcm_t = _matmul_pcm_t(card_embeddings, kernel, bias_row, metapath)
    table = (
        pcm_t.reshape(_NC, _HALF, _D).transpose(0, 2, 1).reshape(_NC * _D, _HALF)
    )
    cols = pool_cols.astype(jnp.int32).reshape(_NS, _NCHUNK, _CHUNK)
    rows = pool_rows.astype(jnp.int32).reshape(_NS, _NCHUNK, _CHUNK)
    vals = pool_values.reshape(_NS, _NNZ_PER)
    out3 = _sc_call(table, cols, rows, vals)
    return out3.transpose(1, 0, 2).reshape(_B, _M)


# speedup vs baseline: 6.0665x; 1.0480x over previous
import functools

import jax
import jax.numpy as jnp
from jax import lax
from jax.experimental import pallas as pl
from jax.experimental.pallas import tpu as pltpu
from jax.experimental.pallas import tpu_sc as plsc

_B = 4096
_D = 4096
_C = 20000
_E = 256
_M = 128
_NNZ = 163840

_CB = 1000
_NCB = _C // _CB

_NC = 2
_NS = 16
_HALF = _M // _NC
_NNZ_PER = _NNZ // _NS
_CHUNK = 128
_NCHUNK = _NNZ_PER // _CHUNK
_BROWS = _B // _NS


def _mm_body(ce_ref, w_ref, b_ref, mp_ref, out_ref):
    pce = jnp.dot(ce_ref[...], w_ref[...], preferred_element_type=jnp.float32)
    pce = pce + b_ref[...]
    pce = jax.nn.swish(pce)
    pce_t = pce.T
    upd = jnp.dot(pce_t, mp_ref[...], preferred_element_type=jnp.float32)

    @pl.when(pl.program_id(0) == 0)
    def _():
        out_ref[...] = upd

    @pl.when(pl.program_id(0) != 0)
    def _():
        out_ref[...] += upd


def _matmul_pcm_t(card_embeddings, w, bias_row, metapath):
    return pl.pallas_call(
        _mm_body,
        grid=(_NCB,),
        in_specs=[
            pl.BlockSpec((_CB, _E), lambda i: (i, 0)),
            pl.BlockSpec((_E, _M), lambda i: (0, 0)),
            pl.BlockSpec((1, _M), lambda i: (0, 0)),
            pl.BlockSpec((_CB, _D), lambda i: (i, 0)),
        ],
        out_specs=pl.BlockSpec((_M, _D), lambda i: (0, 0)),
        out_shape=jax.ShapeDtypeStruct((_M, _D), jnp.float32),
        compiler_params=pltpu.CompilerParams(
            dimension_semantics=("arbitrary",),
        ),
    )(card_embeddings, w, bias_row, metapath)


def _sc_body(table_hbm, cols_hbm, rows_hbm, vals_hbm, out_hbm,
             cols_v, rows_v, vals_v, gb, sb, zb, acc, gsem, ssem0, ssem1):
    cid = lax.axis_index("c")
    sid = lax.axis_index("s")

    pltpu.sync_copy(cols_hbm.at[sid], cols_v)
    pltpu.sync_copy(rows_hbm.at[sid], rows_v)
    pltpu.sync_copy(vals_hbm.at[sid], vals_v)

    offv = jnp.full((16,), cid * _D, jnp.int32)

    def _off_body(j, _):
        for k in range(_CHUNK // 16):
            cols_v[j, pl.ds(k * 16, 16)] = cols_v[j, pl.ds(k * 16, 16)] + offv
        return 0

    lax.fori_loop(0, _NCHUNK, _off_body, 0)

    zv = jnp.zeros((16,), jnp.float32)

    def _zero_body(r, _):
        for k in range(_HALF // 16):
            zb[r, pl.ds(k * 16, 16)] = zv
        return 0

    lax.fori_loop(0, _BROWS, _zero_body, 0)
    pltpu.sync_copy(zb, acc.at[pl.ds(sid * _BROWS, _BROWS)])
    plsc.subcore_barrier()

    iota16 = lax.iota(jnp.int32, 16)

    def _fire(c, b):
        return pltpu.async_copy(table_hbm.at[cols_v.at[c]], gb.at[b], gsem)

    _fire(0, 0)

    def _chunk_pair(i, _):
        for b in (0, 1):
            ssem = ssem0 if b == 0 else ssem1
            c = 2 * i + b
            pltpu.make_async_copy(table_hbm.at[cols_v.at[c]], gb.at[b],
                                  gsem).wait()

            @pl.when(c + 1 < _NCHUNK)
            def _():
                _fire(c + 1, 1 - b)

            @pl.when(c >= 2)
            def _():
                cm2 = jnp.maximum(c - 2, 0)
                pltpu.make_async_copy(sb.at[b], acc.at[rows_v.at[cm2]],
                                      ssem).wait()

            base = c * _CHUNK
            gbb = gb.at[b]
            sbb = sb.at[b]

            @plsc.parallel_loop(0, _CHUNK // 16, 1, unroll=2)
            def _blk(g):
                vg = vals_v[pl.ds(pl.multiple_of(base + g * 16, 16), 16)]
                for l in range(16):
                    v = vg.at[jnp.full((16,), l, jnp.int32)].get(
                        mode="promise_in_bounds")
                    r = g * 16 + l
                    for k in range(_HALF // 16):
                        sbb[r, pl.ds(k * 16, 16)] = gbb[r, pl.ds(k * 16, 16)] * v
            pltpu.async_copy(sbb, acc.at[rows_v.at[c]], ssem, add=True)
        return 0

    lax.fori_loop(0, _NCHUNK // 2, _chunk_pair, 0)
    pltpu.make_async_copy(sb.at[0], acc.at[rows_v.at[_NCHUNK - 2]],
                          ssem0).wait()
    pltpu.make_async_copy(sb.at[1], acc.at[rows_v.at[_NCHUNK - 1]],
                          ssem1).wait()
    plsc.subcore_barrier()

    pltpu.sync_copy(acc.at[pl.ds(sid * _BROWS, _BROWS)],
                    out_hbm.at[pl.ds(sid * _BROWS, _BROWS),
                               pl.ds(cid * _HALF, _HALF)])


_sc_call = functools.partial(
    pl.kernel,
    out_type=jax.ShapeDtypeStruct((_B, _M), jnp.float32),
    mesh=plsc.VectorSubcoreMesh(core_axis_name="c", subcore_axis_name="s"),
    scratch_types=[
        pltpu.VMEM((_NCHUNK, _CHUNK), jnp.int32),
        pltpu.VMEM((_NCHUNK, _CHUNK), jnp.int32),
        pltpu.VMEM((_NNZ_PER,), jnp.float32),
        pltpu.VMEM((2, _CHUNK, _HALF), jnp.float32),
        pltpu.VMEM((2, _CHUNK, _HALF), jnp.float32),
        pltpu.VMEM((_BROWS, _HALF), jnp.float32),
        pltpu.VMEM_SHARED((_B, _HALF), jnp.float32),
        pltpu.SemaphoreType.DMA,
        pltpu.SemaphoreType.DMA,
        pltpu.SemaphoreType.DMA,
    ],
    compiler_params=pltpu.CompilerParams(use_tc_tiling_on_sc=False),
)(_sc_body)


def kernel(pool_values, card_embeddings, metapath, kernel, bias, pool_rows, pool_cols):
    bias_row = bias.reshape(1, _M)
    pcm_t = _matmul_pcm_t(card_embeddings, kernel, bias_row, metapath)
    table = (
        pcm_t.reshape(_NC, _HALF, _D).transpose(0, 2, 1).reshape(_NC * _D, _HALF)
    )
    cols = pool_cols.astype(jnp.int32).reshape(_NS, _NCHUNK, _CHUNK)
    rows = pool_rows.astype(jnp.int32).reshape(_NS, _NCHUNK, _CHUNK)
    vals = pool_values.reshape(_NS, _NNZ_PER)
    return _sc_call(table, cols, rows, vals)


# speedup vs baseline: 6.4858x; 1.0691x over previous
import functools

import jax
import jax.numpy as jnp
from jax import lax
from jax.experimental import pallas as pl
from jax.experimental.pallas import tpu as pltpu
from jax.experimental.pallas import tpu_sc as plsc

_B = 4096
_D = 4096
_C = 20000
_E = 256
_M = 128
_NNZ = 163840

_CB = 1000
_NCB = _C // _CB

_NC = 2
_NS = 16
_NW = _NC * _NS
_NNZ_PER = _NNZ // _NW
_CHUNK = 128
_NCHUNK = _NNZ_PER // _CHUNK
_BROWS = _B // _NS
_ZROWS = 64


def _mm_body(ce_ref, w_ref, b_ref, mp_ref, out_ref):
    pce = jnp.dot(ce_ref[...], w_ref[...], preferred_element_type=jnp.float32)
    pce = pce + b_ref[...]
    pce = jax.nn.swish(pce)
    upd = lax.dot_general(mp_ref[...], pce, (((0,), (0,)), ((), ())),
                          preferred_element_type=jnp.float32)

    @pl.when(pl.program_id(0) == 0)
    def _():
        out_ref[...] = upd

    @pl.when(pl.program_id(0) != 0)
    def _():
        out_ref[...] += upd


def _matmul_pcm(card_embeddings, w, bias_row, metapath):
    return pl.pallas_call(
        _mm_body,
        grid=(_NCB,),
        in_specs=[
            pl.BlockSpec((_CB, _E), lambda i: (i, 0)),
            pl.BlockSpec((_E, _M), lambda i: (0, 0)),
            pl.BlockSpec((1, _M), lambda i: (0, 0)),
            pl.BlockSpec((_CB, _D), lambda i: (i, 0)),
        ],
        out_specs=pl.BlockSpec((_D, _M), lambda i: (0, 0)),
        out_shape=jax.ShapeDtypeStruct((_D, _M), jnp.float32),
        compiler_params=pltpu.CompilerParams(
            dimension_semantics=("arbitrary",),
        ),
    )(card_embeddings, w, bias_row, metapath)


def _sc_body(table_hbm, cols_hbm, rows_hbm, vals_hbm, out_hbm,
             cols_v, rows_v, vals_v, gb, sb, zb, acc, gsem, ssem0, ssem1):
    cid = lax.axis_index("c")
    sid = lax.axis_index("s")
    wid = cid * _NS + sid

    pltpu.sync_copy(cols_hbm.at[wid], cols_v)
    pltpu.sync_copy(rows_hbm.at[wid], rows_v)
    pltpu.sync_copy(vals_hbm.at[wid], vals_v)

    zv = jnp.zeros((16,), jnp.float32)

    def _zero_body(r, _):
        for k in range(_M // 16):
            zb[r, pl.ds(k * 16, 16)] = zv
        return 0

    lax.fori_loop(0, _ZROWS, _zero_body, 0)
    for q in range(_BROWS // _ZROWS):
        pltpu.sync_copy(zb, acc.at[pl.ds(sid * _BROWS + q * _ZROWS, _ZROWS)])
    plsc.subcore_barrier()

    iota16 = lax.iota(jnp.int32, 16)

    def _fire(c, b):
        return pltpu.async_copy(table_hbm.at[cols_v.at[c]], gb.at[b], gsem)

    _fire(0, 0)

    def _chunk_pair(i, _):
        for b in (0, 1):
            ssem = ssem0 if b == 0 else ssem1
            c = 2 * i + b
            pltpu.make_async_copy(table_hbm.at[cols_v.at[c]], gb.at[b],
                                  gsem).wait()

            @pl.when(c + 1 < _NCHUNK)
            def _():
                _fire(c + 1, 1 - b)

            @pl.when(c >= 2)
            def _():
                cm2 = jnp.maximum(c - 2, 0)
                pltpu.make_async_copy(sb.at[b], acc.at[rows_v.at[cm2]],
                                      ssem).wait()

            base = c * _CHUNK
            gbb = gb.at[b]
            sbb = sb.at[b]

            @plsc.parallel_loop(0, _CHUNK // 16, 1, unroll=2)
            def _blk(g):
                vg = vals_v[pl.ds(pl.multiple_of(base + g * 16, 16), 16)]
                for l in range(16):
                    v = vg.at[jnp.full((16,), l, jnp.int32)].get(
                        mode="promise_in_bounds")
                    r = g * 16 + l
                    for k in range(_M // 16):
                        sbb[r, pl.ds(k * 16, 16)] = gbb[r, pl.ds(k * 16, 16)] * v
            pltpu.async_copy(sbb, acc.at[rows_v.at[c]], ssem, add=True)
        return 0

    lax.fori_loop(0, _NCHUNK // 2, _chunk_pair, 0)
    pltpu.make_async_copy(sb.at[0], acc.at[rows_v.at[_NCHUNK - 2]],
                          ssem0).wait()
    pltpu.make_async_copy(sb.at[1], acc.at[rows_v.at[_NCHUNK - 1]],
                          ssem1).wait()
    plsc.subcore_barrier()

    pltpu.sync_copy(acc.at[pl.ds(sid * _BROWS, _BROWS)],
                    out_hbm.at[cid, pl.ds(sid * _BROWS, _BROWS)])


_sc_call = functools.partial(
    pl.kernel,
    out_type=jax.ShapeDtypeStruct((_NC, _B, _M), jnp.float32),
    mesh=plsc.VectorSubcoreMesh(core_axis_name="c", subcore_axis_name="s"),
    scratch_types=[
        pltpu.VMEM((_NCHUNK, _CHUNK), jnp.int32),
        pltpu.VMEM((_NCHUNK, _CHUNK), jnp.int32),
        pltpu.VMEM((_NNZ_PER,), jnp.float32),
        pltpu.VMEM((2, _CHUNK, _M), jnp.float32),
        pltpu.VMEM((2, _CHUNK, _M), jnp.float32),
        pltpu.VMEM((_ZROWS, _M), jnp.float32),
        pltpu.VMEM_SHARED((_B, _M), jnp.float32),
        pltpu.SemaphoreType.DMA,
        pltpu.SemaphoreType.DMA,
        pltpu.SemaphoreType.DMA,
    ],
    compiler_params=pltpu.CompilerParams(use_tc_tiling_on_sc=False),
)(_sc_body)


def _add_body(p_ref, o_ref):
    o_ref[...] = p_ref[0] + p_ref[1]


def _combine(parts):
    nblk = 4
    return pl.pallas_call(
        _add_body,
        grid=(nblk,),
        in_specs=[pl.BlockSpec((_NC, _B // nblk, _M), lambda i: (0, i, 0))],
        out_specs=pl.BlockSpec((_B // nblk, _M), lambda i: (i, 0)),
        out_shape=jax.ShapeDtypeStruct((_B, _M), jnp.float32),
        compiler_params=pltpu.CompilerParams(
            dimension_semantics=("arbitrary",),
        ),
    )(parts)


def kernel(pool_values, card_embeddings, metapath, kernel, bias, pool_rows, pool_cols):
    bias_row = bias.reshape(1, _M)
    table = _matmul_pcm(card_embeddings, kernel, bias_row, metapath)
    cols = pool_cols.astype(jnp.int32).reshape(_NW, _NCHUNK, _CHUNK)
    rows = pool_rows.astype(jnp.int32).reshape(_NW, _NCHUNK, _CHUNK)
    vals = pool_values.reshape(_NW, _NNZ_PER)
    parts = _sc_call(table, cols, rows, vals)
    return _combine(parts)


# speedup vs baseline: 6.6386x; 1.0236x over previous
import functools

import jax
import jax.numpy as jnp
from jax import lax
from jax.experimental import pallas as pl
from jax.experimental.pallas import tpu as pltpu
from jax.experimental.pallas import tpu_sc as plsc

_B = 4096
_D = 4096
_C = 20000
_E = 256
_M = 128
_NNZ = 163840

_CB = 1000
_NCB = _C // _CB

_NC = 2
_NS = 16
_NW = _NC * _NS
_NNZ_PER = _NNZ // _NW
_CHUNK = 128
_NCHUNK = _NNZ_PER // _CHUNK
_BROWS = _B // _NS
_ZROWS = 64


def _mm_body(ce_ref, w_ref, b_ref, mp_ref, out_ref):
    pce = jnp.dot(ce_ref[...], w_ref[...], preferred_element_type=jnp.float32)
    pce = pce + b_ref[...]
    pce = jax.nn.swish(pce)
    upd = lax.dot_general(mp_ref[...], pce, (((0,), (0,)), ((), ())),
                          preferred_element_type=jnp.float32)

    @pl.when(pl.program_id(0) == 0)
    def _():
        out_ref[...] = upd

    @pl.when(pl.program_id(0) != 0)
    def _():
        out_ref[...] += upd


def _matmul_pcm(card_embeddings, w, bias_row, metapath):
    return pl.pallas_call(
        _mm_body,
        grid=(_NCB,),
        in_specs=[
            pl.BlockSpec((_CB, _E), lambda i: (i, 0)),
            pl.BlockSpec((_E, _M), lambda i: (0, 0)),
            pl.BlockSpec((1, _M), lambda i: (0, 0)),
            pl.BlockSpec((_CB, _D), lambda i: (i, 0)),
        ],
        out_specs=pl.BlockSpec((_D, _M), lambda i: (0, 0)),
        out_shape=jax.ShapeDtypeStruct((_D, _M), jnp.float32),
        compiler_params=pltpu.CompilerParams(
            dimension_semantics=("arbitrary",),
        ),
    )(card_embeddings, w, bias_row, metapath)


def _sc_body(table_hbm, cols_hbm, rows_hbm, vals_hbm, out_hbm,
             cols_v, rows_v, vals_v, gb, zb, acc,
             stsem, g0, g1, g2, g3, s0, s1, s2, s3):
    gsems = (g0, g1, g2, g3)
    ssems = (s0, s1, s2, s3)
    cid = lax.axis_index("c")
    sid = lax.axis_index("s")
    wid = cid * _NS + sid

    st0 = pltpu.async_copy(cols_hbm.at[wid], cols_v, stsem)
    st1 = pltpu.async_copy(rows_hbm.at[wid], rows_v, stsem)
    st2 = pltpu.async_copy(vals_hbm.at[wid], vals_v, stsem)

    zv = jnp.zeros((16,), jnp.float32)

    def _zero_body(r, _):
        for k in range(_M // 16):
            zb[r, pl.ds(k * 16, 16)] = zv
        return 0

    lax.fori_loop(0, _ZROWS, _zero_body, 0)
    st0.wait()
    st1.wait()
    st2.wait()

    def _fire(c, b):
        return pltpu.async_copy(table_hbm.at[cols_v.at[c]], gb.at[b],
                                gsems[b])

    _fire(0, 0)
    _fire(1, 1)

    for q in range(_BROWS // _ZROWS):
        pltpu.sync_copy(zb, acc.at[pl.ds(sid * _BROWS + q * _ZROWS, _ZROWS)])
    plsc.subcore_barrier()

    def _chunk_quad(i, _):
        for j in range(4):
            c = 4 * i + j
            jn = (j + 2) % 4
            pltpu.make_async_copy(table_hbm.at[cols_v.at[c]], gb.at[j],
                                  gsems[j]).wait()

            base = c * _CHUNK
            gbb = gb.at[j]

            @plsc.parallel_loop(0, _CHUNK // 16, 1, unroll=2)
            def _blk(g):
                vg = vals_v[pl.ds(pl.multiple_of(base + g * 16, 16), 16)]
                for l in range(16):
                    v = vg.at[jnp.full((16,), l, jnp.int32)].get(
                        mode="promise_in_bounds")
                    r = g * 16 + l
                    for k in range(_M // 16):
                        gbb[r, pl.ds(k * 16, 16)] = gbb[r, pl.ds(k * 16, 16)] * v

            @pl.when(c >= 2)
            def _():
                cm2 = jnp.maximum(c - 2, 0)
                pltpu.make_async_copy(gb.at[jn], acc.at[rows_v.at[cm2]],
                                      ssems[jn]).wait()

            @pl.when(c + 2 < _NCHUNK)
            def _():
                _fire(c + 2, jn)

            pltpu.async_copy(gbb, acc.at[rows_v.at[c]], ssems[j], add=True)
        return 0

    lax.fori_loop(0, _NCHUNK // 4, _chunk_quad, 0)
    pltpu.make_async_copy(gb.at[(_NCHUNK - 2) % 4],
                          acc.at[rows_v.at[_NCHUNK - 2]],
                          ssems[(_NCHUNK - 2) % 4]).wait()
    pltpu.make_async_copy(gb.at[(_NCHUNK - 1) % 4],
                          acc.at[rows_v.at[_NCHUNK - 1]],
                          ssems[(_NCHUNK - 1) % 4]).wait()
    plsc.subcore_barrier()

    pltpu.sync_copy(acc.at[pl.ds(sid * _BROWS, _BROWS)],
                    out_hbm.at[cid, pl.ds(sid * _BROWS, _BROWS)])


_sc_call = functools.partial(
    pl.kernel,
    out_type=jax.ShapeDtypeStruct((_NC, _B, _M), jnp.float32),
    mesh=plsc.VectorSubcoreMesh(core_axis_name="c", subcore_axis_name="s"),
    scratch_types=[
        pltpu.VMEM((_NCHUNK, _CHUNK), jnp.int32),
        pltpu.VMEM((_NCHUNK, _CHUNK), jnp.int32),
        pltpu.VMEM((_NNZ_PER,), jnp.float32),
        pltpu.VMEM((4, _CHUNK, _M), jnp.float32),
        pltpu.VMEM((_ZROWS, _M), jnp.float32),
        pltpu.VMEM_SHARED((_B, _M), jnp.float32),
        pltpu.SemaphoreType.DMA,
        pltpu.SemaphoreType.DMA,
        pltpu.SemaphoreType.DMA,
        pltpu.SemaphoreType.DMA,
        pltpu.SemaphoreType.DMA,
        pltpu.SemaphoreType.DMA,
        pltpu.SemaphoreType.DMA,
        pltpu.SemaphoreType.DMA,
        pltpu.SemaphoreType.DMA,
    ],
    compiler_params=pltpu.CompilerParams(use_tc_tiling_on_sc=False),
)(_sc_body)


def _add_body(p_ref, o_ref):
    o_ref[...] = p_ref[0] + p_ref[1]


def _combine(parts):
    nblk = 4
    return pl.pallas_call(
        _add_body,
        grid=(nblk,),
        in_specs=[pl.BlockSpec((_NC, _B // nblk, _M), lambda i: (0, i, 0))],
        out_specs=pl.BlockSpec((_B // nblk, _M), lambda i: (i, 0)),
        out_shape=jax.ShapeDtypeStruct((_B, _M), jnp.float32),
        compiler_params=pltpu.CompilerParams(
            dimension_semantics=("arbitrary",),
        ),
    )(parts)


def kernel(pool_values, card_embeddings, metapath, kernel, bias, pool_rows, pool_cols):
    bias_row = bias.reshape(1, _M)
    table = _matmul_pcm(card_embeddings, kernel, bias_row, metapath)
    cols = pool_cols.astype(jnp.int32).reshape(_NW, _NCHUNK, _CHUNK)
    rows = pool_rows.astype(jnp.int32).reshape(_NW, _NCHUNK, _CHUNK)
    vals = pool_values.reshape(_NW, _NNZ_PER)
    parts = _sc_call(table, cols, rows, vals)
    return _combine(parts)


# speedup vs baseline: 6.6978x; 1.0089x over previous
import functools

import jax
import jax.numpy as jnp
from jax import lax
from jax.experimental import pallas as pl
from jax.experimental.pallas import tpu as pltpu
from jax.experimental.pallas import tpu_sc as plsc

_B = 4096
_D = 4096
_C = 20000
_E = 256
_M = 128
_NNZ = 163840

_CB = 1000
_NCB = _C // _CB

_NC = 2
_NS = 16
_NW = _NC * _NS
_NNZ_PER = _NNZ // _NW
_CHUNK = 128
_NCHUNK = _NNZ_PER // _CHUNK
_BROWS = _B // _NS
_ZROWS = 64


def _mm_body(ce_ref, w_ref, b_ref, mp_ref, out_ref):
    pce = jnp.dot(ce_ref[...], w_ref[...], preferred_element_type=jnp.float32)
    pce = pce + b_ref[...]
    pce = jax.nn.swish(pce)
    upd = lax.dot_general(mp_ref[...], pce, (((0,), (0,)), ((), ())),
                          preferred_element_type=jnp.float32)

    @pl.when(pl.program_id(0) == 0)
    def _():
        out_ref[...] = upd

    @pl.when(pl.program_id(0) != 0)
    def _():
        out_ref[...] += upd


def _matmul_pcm(card_embeddings, w, bias_row, metapath):
    return pl.pallas_call(
        _mm_body,
        grid=(_NCB,),
        in_specs=[
            pl.BlockSpec((_CB, _E), lambda i: (i, 0)),
            pl.BlockSpec((_E, _M), lambda i: (0, 0)),
            pl.BlockSpec((1, _M), lambda i: (0, 0)),
            pl.BlockSpec((_CB, _D), lambda i: (i, 0)),
        ],
        out_specs=pl.BlockSpec((_D, _M), lambda i: (0, 0)),
        out_shape=jax.ShapeDtypeStruct((_D, _M), jnp.float32),
        compiler_params=pltpu.CompilerParams(
            dimension_semantics=("arbitrary",),
        ),
    )(card_embeddings, w, bias_row, metapath)


def _sc_body(table_hbm, cols_hbm, rows_hbm, vals_hbm, out_hbm,
             cols_v, rows_v, vals_v, gb, acc,
             stsem, g0, g1, g2, g3, s0, s1, s2, s3):
    gsems = (g0, g1, g2, g3)
    ssems = (s0, s1, s2, s3)
    cid = lax.axis_index("c")
    sid = lax.axis_index("s")
    wid = cid * _NS + sid

    st0 = pltpu.async_copy(cols_hbm.at[wid], cols_v, stsem)
    st1 = pltpu.async_copy(rows_hbm.at[wid], rows_v, stsem)
    st2 = pltpu.async_copy(vals_hbm.at[wid], vals_v, stsem)

    zv = jnp.zeros((16,), jnp.float32)
    zbb = gb.at[3]

    def _zero_body(r, _):
        for k in range(_M // 16):
            zbb[r, pl.ds(k * 16, 16)] = zv
        return 0

    lax.fori_loop(0, _ZROWS, _zero_body, 0)
    st0.wait()
    st1.wait()
    st2.wait()

    def _fire(c, b):
        return pltpu.async_copy(table_hbm.at[cols_v.at[c]], gb.at[b],
                                gsems[b])

    _fire(0, 0)
    _fire(1, 1)

    zc = [pltpu.async_copy(gb.at[3, pl.ds(0, _ZROWS)],
                           acc.at[pl.ds(sid * _BROWS + q * _ZROWS, _ZROWS)],
                           stsem)
          for q in range(_BROWS // _ZROWS)]
    for d in zc:
        d.wait()
    plsc.subcore_barrier()

    def _chunk_quad(i, _):
        for j in range(4):
            c = 4 * i + j
            jn = (j + 2) % 4
            pltpu.make_async_copy(table_hbm.at[cols_v.at[c]], gb.at[j],
                                  gsems[j]).wait()

            base = c * _CHUNK
            gbb = gb.at[j]

            @plsc.parallel_loop(0, _CHUNK // 16, 1, unroll=4)
            def _blk(g):
                vg = vals_v[pl.ds(pl.multiple_of(base + g * 16, 16), 16)]
                for l in range(16):
                    v = vg.at[jnp.full((16,), l, jnp.int32)].get(
                        mode="promise_in_bounds")
                    r = g * 16 + l
                    for k in range(_M // 16):
                        gbb[r, pl.ds(k * 16, 16)] = gbb[r, pl.ds(k * 16, 16)] * v

            @pl.when(c >= 2)
            def _():
                cm2 = jnp.maximum(c - 2, 0)
                pltpu.make_async_copy(gb.at[jn], acc.at[rows_v.at[cm2]],
                                      ssems[jn]).wait()

            @pl.when(c + 2 < _NCHUNK)
            def _():
                _fire(c + 2, jn)

            pltpu.async_copy(gbb, acc.at[rows_v.at[c]], ssems[j], add=True)
        return 0

    lax.fori_loop(0, _NCHUNK // 4, _chunk_quad, 0)
    pltpu.make_async_copy(gb.at[(_NCHUNK - 2) % 4],
                          acc.at[rows_v.at[_NCHUNK - 2]],
                          ssems[(_NCHUNK - 2) % 4]).wait()
    pltpu.make_async_copy(gb.at[(_NCHUNK - 1) % 4],
                          acc.at[rows_v.at[_NCHUNK - 1]],
                          ssems[(_NCHUNK - 1) % 4]).wait()
    plsc.subcore_barrier()

    pltpu.sync_copy(acc.at[pl.ds(sid * _BROWS, _BROWS)],
                    out_hbm.at[cid, pl.ds(sid * _BROWS, _BROWS)])


_sc_call = functools.partial(
    pl.kernel,
    out_type=jax.ShapeDtypeStruct((_NC, _B, _M), jnp.float32),
    mesh=plsc.VectorSubcoreMesh(core_axis_name="c", subcore_axis_name="s"),
    scratch_types=[
        pltpu.VMEM((_NCHUNK, _CHUNK), jnp.int32),
        pltpu.VMEM((_NCHUNK, _CHUNK), jnp.int32),
        pltpu.VMEM((_NNZ_PER,), jnp.float32),
        pltpu.VMEM((4, _CHUNK, _M), jnp.float32),
        pltpu.VMEM_SHARED((_B, _M), jnp.float32),
        pltpu.SemaphoreType.DMA,
        pltpu.SemaphoreType.DMA,
        pltpu.SemaphoreType.DMA,
        pltpu.SemaphoreType.DMA,
        pltpu.SemaphoreType.DMA,
        pltpu.SemaphoreType.DMA,
        pltpu.SemaphoreType.DMA,
        pltpu.SemaphoreType.DMA,
        pltpu.SemaphoreType.DMA,
    ],
    compiler_params=pltpu.CompilerParams(use_tc_tiling_on_sc=False),
)(_sc_body)


def _add_body(p_ref, o_ref):
    o_ref[...] = p_ref[0] + p_ref[1]


def _combine(parts):
    nblk = 4
    return pl.pallas_call(
        _add_body,
        grid=(nblk,),
        in_specs=[pl.BlockSpec((_NC, _B // nblk, _M), lambda i: (0, i, 0))],
        out_specs=pl.BlockSpec((_B // nblk, _M), lambda i: (i, 0)),
        out_shape=jax.ShapeDtypeStruct((_B, _M), jnp.float32),
        compiler_params=pltpu.CompilerParams(
            dimension_semantics=("arbitrary",),
        ),
    )(parts)


def kernel(pool_values, card_embeddings, metapath, kernel, bias, pool_rows, pool_cols):
    bias_row = bias.reshape(1, _M)
    table = _matmul_pcm(card_embeddings, kernel, bias_row, metapath)
    cols = pool_cols.astype(jnp.int32).reshape(_NW, _NCHUNK, _CHUNK)
    rows = pool_rows.astype(jnp.int32).reshape(_NW, _NCHUNK, _CHUNK)
    vals = pool_values.reshape(_NW, _NNZ_PER)
    parts = _sc_call(table, cols, rows, vals)
    return _combine(parts)
